# baseline (speedup 1.0000x reference)
import functools

import jax
import jax.numpy as jnp
from jax import lax
from jax.experimental import pallas as pl
from jax.experimental.pallas import tpu as pltpu
from jax.experimental.pallas import tpu_sc as plsc

_MASK_RATE = 0.3
_NC = 2
_NS = 16
_NW = _NC * _NS
_C = 128
_SPAN = 80
_ZR = 125
_G = 16


def _sc_mesh():
    return plsc.VectorSubcoreMesh(core_axis_name="c", subcore_axis_name="s")


def _span_plan(R):
    n_full = R // _SPAN
    rem = R % _SPAN
    assert n_full < _NW or (n_full == _NW and rem == 0)
    return n_full, rem


def _fill_rows(ref, nrows, ncols, value):
    vec = jnp.full((16,), value, jnp.float32)
    cols = ncols // 16

    def body(i, _):
        for j in range(cols):
            ref[i, pl.ds(j * 16, 16)] = vec
        return 0

    lax.fori_loop(0, nrows, body, 0)


def _deg_body(src2_hbm, dst2_hbm, out_hbm, dacc0, dacc1, h0, h1,
              hstage0, hstage1, sidx16, didx16, ibuf, zbuf, dsem):
    c = lax.axis_index("c")
    s = lax.axis_index("s")
    w = c * _NS + s
    N = out_hbm.shape[2]
    NP = dacc0.shape[0]
    n_full, rem = _span_plan(src2_hbm.shape[0])
    assert n_full == _NW - 1 and rem > 0 and _SPAN % _G == 0
    ngroups = jnp.where(w < n_full, _SPAN // _G, rem // _G)
    gtail = rem % _G
    rpsp = NP // _NS

    ones16 = jnp.full((16,), 1.0, jnp.float32)
    zero16 = jnp.zeros((16,), jnp.float32)

    def zh(i, carry):
        h0[pl.ds(i * 16, 16)] = zero16
        h1[pl.ds(i * 16, 16)] = zero16
        return carry

    lax.fori_loop(0, NP // 16, zh, 0)
    _fill_rows(zbuf, _C, 16, 0.0)
    for k in range(rpsp // _C):
        base = s * rpsp + k * _C
        pltpu.sync_copy(zbuf, dacc0.at[pl.ds(base, _C)])
        pltpu.sync_copy(zbuf, dacc1.at[pl.ds(base, _C)])
    plsc.subcore_barrier()

    def group_body(rowbase, cnt):
        sd = pltpu.async_copy(src2_hbm.at[pl.ds(rowbase, cnt)],
                              sidx16.at[pl.ds(0, cnt)], dsem)
        pltpu.sync_copy(dst2_hbm.at[pl.ds(rowbase, cnt)],
                        didx16.at[pl.ds(0, cnt)])
        sd.wait()
        for u in range(cnt):
            for v in range(_C // 16):
                plsc.addupdate_scatter(h0, [sidx16[u, pl.ds(v * 16, 16)]],
                                       ones16)
                plsc.addupdate_scatter(h1, [didx16[u, pl.ds(v * 16, 16)]],
                                       ones16)

    def group(g, carry):
        group_body(w * _SPAN + g * _G, _G)
        return carry

    lax.fori_loop(0, ngroups, group, 0)
    if gtail:
        @pl.when(w == n_full)
        def _():
            group_body(n_full * _SPAN + (rem // _G) * _G, gtail)

    for m in range(NP // 16 // _C):
        cb = m * _C

        def repack(r, carry, cb=cb):
            hstage0[r, :] = h0[pl.ds((cb + r) * 16, 16)]
            hstage1[r, :] = h1[pl.ds((cb + r) * 16, 16)]
            return carry

        lax.fori_loop(0, _C, repack, 0)
        for j in range(_C // 16):
            ibuf[pl.ds(j * 16, 16)] = (lax.iota(jnp.int32, 16)
                                       + (cb + j * 16))
        pltpu.sync_copy(hstage0, dacc0.at[ibuf], add=True)
        pltpu.sync_copy(hstage1, dacc1.at[ibuf], add=True)
    plsc.subcore_barrier()
    ra = (N // _NS) & ~7
    row0 = s * ra
    pltpu.sync_copy(dacc0.at[pl.ds(row0, ra)], out_hbm.at[c, 0, pl.ds(row0, ra)])
    pltpu.sync_copy(dacc1.at[pl.ds(row0, ra)], out_hbm.at[c, 1, pl.ds(row0, ra)])
    tail = N - ra * _NS
    if tail:
        @pl.when(s == _NS - 1)
        def _():
            t0 = ra * _NS
            pltpu.sync_copy(dacc0.at[pl.ds(t0, tail)],
                            out_hbm.at[c, 0, pl.ds(t0, tail)])
            pltpu.sync_copy(dacc1.at[pl.ds(t0, tail)],
                            out_hbm.at[c, 1, pl.ds(t0, tail)])


def _mp_body(hs_hbm, src2_hbm, dst2_hbm, out_hbm, acc, r0, r1,
             sidx16, didx16, gs0, gs1):
    rows = [r0, r1]
    gsem = [gs0, gs1]
    c = lax.axis_index("c")
    s = lax.axis_index("s")
    w = c * _NS + s
    R = src2_hbm.shape[0]
    N, D = hs_hbm.shape
    n_full, rem = _span_plan(R)
    assert n_full == _NW - 1 and rem > 0
    assert _SPAN % _G == 0
    ngroups = jnp.where(w < n_full, _SPAN // _G, rem // _G)
    tail = rem % _G
    rps = N // _NS

    _fill_rows(r0, _C, D, 0.0)
    zds = [pltpu.async_copy(r0.at[pl.ds(0, _ZR)],
                            acc.at[pl.ds(s * rps + k * _ZR, _ZR)], gs0)
           for k in range(rps // _ZR)]
    for d in zds:
        d.wait()
    plsc.subcore_barrier()

    def group_body(rowbase, cnt):
        sd = pltpu.async_copy(src2_hbm.at[pl.ds(rowbase, cnt)],
                              sidx16.at[pl.ds(0, cnt)], gs0)
        pltpu.sync_copy(dst2_hbm.at[pl.ds(rowbase, cnt)],
                        didx16.at[pl.ds(0, cnt)])
        sd.wait()
        descs = {}
        for u in range(min(2, cnt)):
            descs[u] = pltpu.async_copy(
                hs_hbm.at[sidx16.at[u]], rows[u % 2], gsem[u % 2])
        for u in range(cnt):
            descs[u].wait()
            pltpu.sync_copy(rows[u % 2], acc.at[didx16.at[u]], add=True)
            if u + 2 < cnt:
                descs[u + 2] = pltpu.async_copy(
                    hs_hbm.at[sidx16.at[u + 2]], rows[u % 2], gsem[u % 2])

    def group(g, carry):
        group_body(w * _SPAN + g * _G, _G)
        return carry

    lax.fori_loop(0, ngroups, group, 0)
    if tail:
        @pl.when(w == n_full)
        def _():
            group_body(n_full * _SPAN + (rem // _G) * _G, tail)

    plsc.subcore_barrier()
    ra = (N // _NS) & ~7
    row0 = s * ra
    pltpu.sync_copy(acc.at[pl.ds(row0, ra)], out_hbm.at[c, pl.ds(row0, ra)])
    tail = N - ra * _NS
    if tail:
        @pl.when(s == _NS - 1)
        def _():
            t0 = ra * _NS
            pltpu.sync_copy(acc.at[pl.ds(t0, tail)],
                            out_hbm.at[c, pl.ds(t0, tail)])


def _sc_degrees(src2, dst2, N):
    npad = 16 * _C * ((N + 16 * _C - 1) // (16 * _C))
    k = pl.kernel(
        _deg_body,
        out_type=jax.ShapeDtypeStruct((_NC, 2, N, 16), jnp.float32),
        mesh=_sc_mesh(),
        scratch_types=[
            pltpu.VMEM_SHARED((npad, 16), jnp.float32),
            pltpu.VMEM_SHARED((npad, 16), jnp.float32),
            pltpu.VMEM((npad,), jnp.float32),
            pltpu.VMEM((npad,), jnp.float32),
            pltpu.VMEM((_C, 16), jnp.float32),
            pltpu.VMEM((_C, 16), jnp.float32),
            pltpu.VMEM((_G, _C), jnp.int32),
            pltpu.VMEM((_G, _C), jnp.int32),
            pltpu.VMEM((_C,), jnp.int32),
            pltpu.VMEM((_C, 16), jnp.float32),
            pltpu.SemaphoreType.DMA,
        ],
        compiler_params=pltpu.CompilerParams(needs_layout_passes=False),
    )
    return k(src2, dst2)


def _sc_message_pass(hs, src2, dst2):
    N, D = hs.shape
    k = pl.kernel(
        _mp_body,
        out_type=jax.ShapeDtypeStruct((_NC, N, D), jnp.float32),
        mesh=_sc_mesh(),
        scratch_types=[
            pltpu.VMEM_SHARED((N, D), jnp.float32),
            pltpu.VMEM((_C, D), jnp.float32),
            pltpu.VMEM((_C, D), jnp.float32),
            pltpu.VMEM((_G, _C), jnp.int32),
            pltpu.VMEM((_G, _C), jnp.int32),
            pltpu.SemaphoreType.DMA,
            pltpu.SemaphoreType.DMA,
        ],
    )
    return k(hs, src2, dst2)


def _dot(a, w):
    return jnp.dot(a, w, preferred_element_type=jnp.float32,
                   precision=lax.Precision.HIGHEST)


def _tc_prep_body(x_ref, mk_ref, tok_ref, do0, do1, di0, di1,
                  h1s_ref, dvo_ref, dvi_ref):
    deg_o = do0[:, :1] + do1[:, :1]
    deg_i = di0[:, :1] + di1[:, :1]
    dvo = lax.rsqrt(jnp.maximum(deg_o, 1.0))
    dvi = lax.rsqrt(jnp.maximum(deg_i, 1.0))
    m = mk_ref[...]
    ux = x_ref[...] * (1.0 - m) + m * tok_ref[...]
    shape = ux.shape
    dvo_b = jnp.broadcast_to(dvo, shape)
    h1s_ref[...] = ux * dvo_b
    dvo_ref[...] = dvo_b
    dvi_ref[...] = jnp.broadcast_to(dvi, shape)


def _tc_layer1_body(mp_ref, dvi_ref, dvo_ref, w_ref, b_ref, out_ref):
    m = (mp_ref[0] + mp_ref[1]) * dvi_ref[...]
    y = _dot(m, w_ref[...]) + b_ref[...]
    out_ref[...] = jnp.maximum(y, 0.0) * dvo_ref[...]


def _tc_layer2_body(mp_ref, dvi_ref, dvo_ref, w2_ref, b2_ref, we_ref, mk_ref,
                    out_ref):
    m = (mp_ref[0] + mp_ref[1]) * dvi_ref[...]
    h = jnp.maximum(_dot(m, w2_ref[...]) + b2_ref[...], 0.0)
    rep = _dot(h, we_ref[...])
    out_ref[...] = rep * (1.0 - mk_ref[...]) * dvo_ref[...]


def _tc_loss_body(inv_nmask, mp_ref, dvi_ref, wd_ref, bd_ref, x_ref, mk_ref,
                  out_ref):
    i = pl.program_id(0)
    m = (mp_ref[0] + mp_ref[1]) * dvi_ref[...]
    recon = _dot(m, wd_ref[...]) + bd_ref[...]
    u = x_ref[...]
    duv = jnp.sum(u * recon, axis=1, keepdims=True)
    nu = jnp.sqrt(jnp.sum(u * u, axis=1, keepdims=True))
    nv = jnp.sqrt(jnp.sum(recon * recon, axis=1, keepdims=True))
    cos = duv / ((nu + 1e-8) * (nv + 1e-8))
    t = 1.0 - cos
    part = jnp.sum(t * t * mk_ref[:, :1]) * inv_nmask

    @pl.when(i == 0)
    def _():
        out_ref[0, 0] = 0.0

    out_ref[0, 0] += part


def _row_spec(B, w):
    return pl.BlockSpec((B, w), lambda i: (i, 0))


def _const_spec(shape):
    return pl.BlockSpec(shape, lambda i: tuple(0 for _ in shape))


def kernel(x, edge_index, W_enc1, b_enc1, W_enc2, b_enc2, W_e2d, W_dec, b_dec,
           mask_token):
    N, D = x.shape
    E = edge_index.shape[1]
    src1 = edge_index[0]
    dst1 = edge_index[1]
    src2 = src1.reshape(E // _C, _C)
    dst2 = dst1.reshape(E // _C, _C)

    num_mask = int(_MASK_RATE * N)
    mask_nodes = jax.random.permutation(jax.random.key(42), N)[:num_mask]
    maskf = jnp.zeros((N, 1), jnp.float32).at[mask_nodes].set(1.0)
    maskf = jnp.asarray(jnp.broadcast_to(maskf, (N, D)))

    b1 = b_enc1.reshape(1, D)
    b2 = b_enc2.reshape(1, D)
    bd = b_dec.reshape(1, D)

    B = 5000
    grid = (N // B,)
    mp_spec = pl.BlockSpec((_NC, B, D), lambda i: (0, i, 0))

    degp = _sc_degrees(src2, dst2, N)

    h1s, dvo, dvi = pl.pallas_call(
        _tc_prep_body,
        grid=grid,
        in_specs=[
            _row_spec(B, D), _row_spec(B, D), _const_spec((1, D)),
            _row_spec(B, 16), _row_spec(B, 16),
            _row_spec(B, 16), _row_spec(B, 16),
        ],
        out_specs=[_row_spec(B, D)] * 3,
        out_shape=[jax.ShapeDtypeStruct((N, D), jnp.float32)] * 3,
    )(x, maskf, mask_token, degp[0, 0], degp[1, 0], degp[0, 1], degp[1, 1])

    mp1 = _sc_message_pass(h1s, src2, dst2)

    h2s = pl.pallas_call(
        _tc_layer1_body,
        grid=grid,
        in_specs=[mp_spec, _row_spec(B, D), _row_spec(B, D),
                  _const_spec((D, D)), _const_spec((1, D))],
        out_specs=_row_spec(B, D),
        out_shape=jax.ShapeDtypeStruct((N, D), jnp.float32),
    )(mp1, dvi, dvo, W_enc1, b1)

    mp2 = _sc_message_pass(h2s, src2, dst2)

    h3s = pl.pallas_call(
        _tc_layer2_body,
        grid=grid,
        in_specs=[mp_spec, _row_spec(B, D), _row_spec(B, D),
                  _const_spec((D, D)), _const_spec((1, D)),
                  _const_spec((D, D)), _row_spec(B, D)],
        out_specs=_row_spec(B, D),
        out_shape=jax.ShapeDtypeStruct((N, D), jnp.float32),
    )(mp2, dvi, dvo, W_enc2, b2, W_e2d, maskf)

    mp3 = _sc_message_pass(h3s, src2, dst2)

    out = pl.pallas_call(
        functools.partial(_tc_loss_body, 1.0 / num_mask),
        grid=grid,
        in_specs=[mp_spec, _row_spec(B, D), _const_spec((D, D)),
                  _const_spec((1, D)), _row_spec(B, D), _row_spec(B, D)],
        out_specs=pl.BlockSpec(memory_space=pltpu.SMEM),
        out_shape=jax.ShapeDtypeStruct((1, 1), jnp.float32),
    )(mp3, dvi, W_dec, bd, x, maskf)

    return out[0, 0]


# speedup vs baseline: 1.0433x; 1.0433x over previous
import functools

import jax
import jax.numpy as jnp
from jax import lax
from jax.experimental import pallas as pl
from jax.experimental.pallas import tpu as pltpu
from jax.experimental.pallas import tpu_sc as plsc

_MASK_RATE = 0.3
_NC = 2
_NS = 16
_NW = _NC * _NS
_C = 128
_SPAN = 80
_ZR = 125
_G = 16


def _sc_mesh():
    return plsc.VectorSubcoreMesh(core_axis_name="c", subcore_axis_name="s")


def _span_plan(R):
    n_full = R // _SPAN
    rem = R % _SPAN
    assert n_full < _NW or (n_full == _NW and rem == 0)
    return n_full, rem


def _fill_rows(ref, nrows, ncols, value):
    vec = jnp.full((16,), value, jnp.float32)
    cols = ncols // 16

    def body(i, _):
        for j in range(cols):
            ref[i, pl.ds(j * 16, 16)] = vec
        return 0

    lax.fori_loop(0, nrows, body, 0)


def _deg_body(src2_hbm, dst2_hbm, out_hbm, dacc0, dacc1, h0, h1,
              hstage0, hstage1, sidx16, didx16, ibuf, zbuf, dsem):
    c = lax.axis_index("c")
    s = lax.axis_index("s")
    w = c * _NS + s
    N = out_hbm.shape[2]
    NP = dacc0.shape[0]
    n_full, rem = _span_plan(src2_hbm.shape[0])
    assert n_full == _NW - 1 and rem > 0 and _SPAN % _G == 0
    ngroups = jnp.where(w < n_full, _SPAN // _G, rem // _G)
    gtail = rem % _G
    rpsp = NP // _NS

    ones16 = jnp.full((16,), 1.0, jnp.float32)
    zero16 = jnp.zeros((16,), jnp.float32)

    def zh(i, carry):
        h0[pl.ds(i * 16, 16)] = zero16
        h1[pl.ds(i * 16, 16)] = zero16
        return carry

    lax.fori_loop(0, NP // 16, zh, 0)
    _fill_rows(zbuf, _C, 16, 0.0)
    for k in range(rpsp // _C):
        base = s * rpsp + k * _C
        pltpu.sync_copy(zbuf, dacc0.at[pl.ds(base, _C)])
        pltpu.sync_copy(zbuf, dacc1.at[pl.ds(base, _C)])
    plsc.subcore_barrier()

    def group_body(rowbase, cnt):
        sd = pltpu.async_copy(src2_hbm.at[pl.ds(rowbase, cnt)],
                              sidx16.at[pl.ds(0, cnt)], dsem)
        pltpu.sync_copy(dst2_hbm.at[pl.ds(rowbase, cnt)],
                        didx16.at[pl.ds(0, cnt)])
        sd.wait()
        for u in range(cnt):
            for v in range(_C // 16):
                plsc.addupdate_scatter(h0, [sidx16[u, pl.ds(v * 16, 16)]],
                                       ones16)
                plsc.addupdate_scatter(h1, [didx16[u, pl.ds(v * 16, 16)]],
                                       ones16)

    def group(g, carry):
        group_body(w * _SPAN + g * _G, _G)
        return carry

    lax.fori_loop(0, ngroups, group, 0)
    if gtail:
        @pl.when(w == n_full)
        def _():
            group_body(n_full * _SPAN + (rem // _G) * _G, gtail)

    for m in range(NP // 16 // _C):
        cb = m * _C

        def repack(r, carry, cb=cb):
            hstage0[r, :] = h0[pl.ds((cb + r) * 16, 16)]
            hstage1[r, :] = h1[pl.ds((cb + r) * 16, 16)]
            return carry

        lax.fori_loop(0, _C, repack, 0)
        for j in range(_C // 16):
            ibuf[pl.ds(j * 16, 16)] = (lax.iota(jnp.int32, 16)
                                       + (cb + j * 16))
        pltpu.sync_copy(hstage0, dacc0.at[ibuf], add=True)
        pltpu.sync_copy(hstage1, dacc1.at[ibuf], add=True)
    plsc.subcore_barrier()
    ra = (N // _NS) & ~7
    row0 = s * ra
    pltpu.sync_copy(dacc0.at[pl.ds(row0, ra)], out_hbm.at[c, 0, pl.ds(row0, ra)])
    pltpu.sync_copy(dacc1.at[pl.ds(row0, ra)], out_hbm.at[c, 1, pl.ds(row0, ra)])
    tail = N - ra * _NS
    if tail:
        @pl.when(s == _NS - 1)
        def _():
            t0 = ra * _NS
            pltpu.sync_copy(dacc0.at[pl.ds(t0, tail)],
                            out_hbm.at[c, 0, pl.ds(t0, tail)])
            pltpu.sync_copy(dacc1.at[pl.ds(t0, tail)],
                            out_hbm.at[c, 1, pl.ds(t0, tail)])


def _mp_body(hs_hbm, src2_hbm, dst2_hbm, out_hbm, acc, r0, r1,
             sidx16, didx16, gs0, gs1):
    rows = [r0, r1]
    gsem = [gs0, gs1]
    c = lax.axis_index("c")
    s = lax.axis_index("s")
    w = c * _NS + s
    R = src2_hbm.shape[0]
    N, D = hs_hbm.shape
    n_full, rem = _span_plan(R)
    assert n_full == _NW - 1 and rem > 0
    assert _SPAN % _G == 0
    ngroups = jnp.where(w < n_full, _SPAN // _G, rem // _G)
    tail = rem % _G
    rps = N // _NS

    _fill_rows(r0, _C, D, 0.0)
    zds = [pltpu.async_copy(r0.at[pl.ds(0, _ZR)],
                            acc.at[pl.ds(s * rps + k * _ZR, _ZR)], gs0)
           for k in range(rps // _ZR)]
    for d in zds:
        d.wait()
    plsc.subcore_barrier()

    def group_body(rowbase, cnt):
        sd = pltpu.async_copy(src2_hbm.at[pl.ds(rowbase, cnt)],
                              sidx16.at[pl.ds(0, cnt)], gs0)
        pltpu.sync_copy(dst2_hbm.at[pl.ds(rowbase, cnt)],
                        didx16.at[pl.ds(0, cnt)])
        sd.wait()
        descs = {}
        for u in range(min(2, cnt)):
            descs[u] = pltpu.async_copy(
                hs_hbm.at[sidx16.at[u]], rows[u % 2], gsem[u % 2])
        for u in range(cnt):
            descs[u].wait()
            pltpu.sync_copy(rows[u % 2], acc.at[didx16.at[u]], add=True)
            if u + 2 < cnt:
                descs[u + 2] = pltpu.async_copy(
                    hs_hbm.at[sidx16.at[u + 2]], rows[u % 2], gsem[u % 2])

    def group(g, carry):
        group_body(w * _SPAN + g * _G, _G)
        return carry

    lax.fori_loop(0, ngroups, group, 0)
    if tail:
        @pl.when(w == n_full)
        def _():
            group_body(n_full * _SPAN + (rem // _G) * _G, tail)

    plsc.subcore_barrier()
    ra = (N // _NS) & ~7
    row0 = s * ra
    pltpu.sync_copy(acc.at[pl.ds(row0, ra)], out_hbm.at[c, pl.ds(row0, ra)])
    tail = N - ra * _NS
    if tail:
        @pl.when(s == _NS - 1)
        def _():
            t0 = ra * _NS
            pltpu.sync_copy(acc.at[pl.ds(t0, tail)],
                            out_hbm.at[c, pl.ds(t0, tail)])


def _sc_degrees(src2, dst2, N):
    npad = 16 * _C * ((N + 16 * _C - 1) // (16 * _C))
    k = pl.kernel(
        _deg_body,
        out_type=jax.ShapeDtypeStruct((_NC, 2, N, 16), jnp.float32),
        mesh=_sc_mesh(),
        scratch_types=[
            pltpu.VMEM_SHARED((npad, 16), jnp.float32),
            pltpu.VMEM_SHARED((npad, 16), jnp.float32),
            pltpu.VMEM((npad,), jnp.float32),
            pltpu.VMEM((npad,), jnp.float32),
            pltpu.VMEM((_C, 16), jnp.float32),
            pltpu.VMEM((_C, 16), jnp.float32),
            pltpu.VMEM((_G, _C), jnp.int32),
            pltpu.VMEM((_G, _C), jnp.int32),
            pltpu.VMEM((_C,), jnp.int32),
            pltpu.VMEM((_C, 16), jnp.float32),
            pltpu.SemaphoreType.DMA,
        ],
        compiler_params=pltpu.CompilerParams(needs_layout_passes=False),
    )
    return k(src2, dst2)


def _sc_message_pass(hs, src2, dst2):
    N, D = hs.shape
    k = pl.kernel(
        _mp_body,
        out_type=jax.ShapeDtypeStruct((_NC, N, D), jnp.float32),
        mesh=_sc_mesh(),
        scratch_types=[
            pltpu.VMEM_SHARED((N, D), jnp.float32),
            pltpu.VMEM((_C, D), jnp.float32),
            pltpu.VMEM((_C, D), jnp.float32),
            pltpu.VMEM((_G, _C), jnp.int32),
            pltpu.VMEM((_G, _C), jnp.int32),
            pltpu.SemaphoreType.DMA,
            pltpu.SemaphoreType.DMA,
        ],
    )
    return k(hs, src2, dst2)


def _dot(a, w):
    return jnp.dot(a, w, preferred_element_type=jnp.float32,
                   precision=lax.Precision.HIGHEST)


def _tc_prep_body(x_ref, mk_ref, tok_ref, do0, do1, di0, di1,
                  h1s_ref, dvo_ref, dvi_ref):
    deg_o = do0[:, :1] + do1[:, :1]
    deg_i = di0[:, :1] + di1[:, :1]
    dvo = lax.rsqrt(jnp.maximum(deg_o, 1.0))
    dvi = lax.rsqrt(jnp.maximum(deg_i, 1.0))
    m = mk_ref[...]
    ux = x_ref[...] * (1.0 - m) + m * tok_ref[...]
    shape = ux.shape
    dvo_b = jnp.broadcast_to(dvo, shape)
    h1s_ref[...] = ux * dvo_b
    dvo_ref[...] = dvo_b
    dvi_ref[...] = jnp.broadcast_to(dvi, shape)


def _tc_layer1_body(mp_ref, dvi_ref, dvo_ref, w_ref, b_ref, out_ref):
    m = (mp_ref[0] + mp_ref[1]) * dvi_ref[...]
    y = _dot(m, w_ref[...]) + b_ref[...]
    out_ref[...] = jnp.maximum(y, 0.0) * dvo_ref[...]


def _tc_layer2_body(mp_ref, dvi_ref, dvo_ref, w2_ref, b2_ref, we_ref, mk_ref,
                    out_ref):
    m = (mp_ref[0] + mp_ref[1]) * dvi_ref[...]
    h = jnp.maximum(_dot(m, w2_ref[...]) + b2_ref[...], 0.0)
    rep = _dot(h, we_ref[...])
    out_ref[...] = rep * (1.0 - mk_ref[...]) * dvo_ref[...]


def _tc_loss_body(inv_nmask, mp_ref, dvi_ref, wd_ref, bd_ref, x_ref, mk_ref,
                  out_ref):
    i = pl.program_id(0)
    m = (mp_ref[0] + mp_ref[1]) * dvi_ref[...]
    recon = _dot(m, wd_ref[...]) + bd_ref[...]
    u = x_ref[...]
    duv = jnp.sum(u * recon, axis=1, keepdims=True)
    nu = jnp.sqrt(jnp.sum(u * u, axis=1, keepdims=True))
    nv = jnp.sqrt(jnp.sum(recon * recon, axis=1, keepdims=True))
    cos = duv / ((nu + 1e-8) * (nv + 1e-8))
    t = 1.0 - cos
    part = jnp.sum(t * t * mk_ref[:, :1]) * inv_nmask

    @pl.when(i == 0)
    def _():
        out_ref[0, 0] = 0.0

    out_ref[0, 0] += part


def _row_spec(B, w):
    return pl.BlockSpec((B, w), lambda i: (i, 0))


def _const_spec(shape):
    return pl.BlockSpec(shape, lambda i: tuple(0 for _ in shape))


def kernel(x, edge_index, W_enc1, b_enc1, W_enc2, b_enc2, W_e2d, W_dec, b_dec,
           mask_token):
    N, D = x.shape
    E = edge_index.shape[1]
    src1 = edge_index[0]
    dst1 = edge_index[1]
    src2 = src1.reshape(E // _C, _C)
    dst2 = dst1.reshape(E // _C, _C)

    num_mask = int(_MASK_RATE * N)
    mask_nodes = jax.random.permutation(jax.random.key(42), N)[:num_mask]
    maskf = jnp.zeros((N, 1), jnp.float32).at[mask_nodes].set(1.0)
    maskf = jnp.asarray(jnp.broadcast_to(maskf, (N, D)))

    b1 = b_enc1.reshape(1, D)
    b2 = b_enc2.reshape(1, D)
    bd = b_dec.reshape(1, D)

    B = 2000
    grid = (N // B,)
    mp_spec = pl.BlockSpec((_NC, B, D), lambda i: (0, i, 0))

    degp = _sc_degrees(src2, dst2, N)

    h1s, dvo, dvi = pl.pallas_call(
        _tc_prep_body,
        grid=grid,
        in_specs=[
            _row_spec(B, D), _row_spec(B, D), _const_spec((1, D)),
            _row_spec(B, 16), _row_spec(B, 16),
            _row_spec(B, 16), _row_spec(B, 16),
        ],
        out_specs=[_row_spec(B, D)] * 3,
        out_shape=[jax.ShapeDtypeStruct((N, D), jnp.float32)] * 3,
    )(x, maskf, mask_token, degp[0, 0], degp[1, 0], degp[0, 1], degp[1, 1])

    mp1 = _sc_message_pass(h1s, src2, dst2)

    h2s = pl.pallas_call(
        _tc_layer1_body,
        grid=grid,
        in_specs=[mp_spec, _row_spec(B, D), _row_spec(B, D),
                  _const_spec((D, D)), _const_spec((1, D))],
        out_specs=_row_spec(B, D),
        out_shape=jax.ShapeDtypeStruct((N, D), jnp.float32),
    )(mp1, dvi, dvo, W_enc1, b1)

    mp2 = _sc_message_pass(h2s, src2, dst2)

    h3s = pl.pallas_call(
        _tc_layer2_body,
        grid=grid,
        in_specs=[mp_spec, _row_spec(B, D), _row_spec(B, D),
                  _const_spec((D, D)), _const_spec((1, D)),
                  _const_spec((D, D)), _row_spec(B, D)],
        out_specs=_row_spec(B, D),
        out_shape=jax.ShapeDtypeStruct((N, D), jnp.float32),
    )(mp2, dvi, dvo, W_enc2, b2, W_e2d, maskf)

    mp3 = _sc_message_pass(h3s, src2, dst2)

    out = pl.pallas_call(
        functools.partial(_tc_loss_body, 1.0 / num_mask),
        grid=grid,
        in_specs=[mp_spec, _row_spec(B, D), _const_spec((D, D)),
                  _const_spec((1, D)), _row_spec(B, D), _row_spec(B, D)],
        out_specs=pl.BlockSpec(memory_space=pltpu.SMEM),
        out_shape=jax.ShapeDtypeStruct((1, 1), jnp.float32),
    )(mp3, dvi, W_dec, bd, x, maskf)

    return out[0, 0]


# speedup vs baseline: 1.0441x; 1.0008x over previous
import functools

import jax
import jax.numpy as jnp
from jax import lax
from jax.experimental import pallas as pl
from jax.experimental.pallas import tpu as pltpu
from jax.experimental.pallas import tpu_sc as plsc

_MASK_RATE = 0.3
_NC = 2
_NS = 16
_NW = _NC * _NS
_C = 128
_SPAN = 80
_ZR = 125
_G = 16


def _sc_mesh():
    return plsc.VectorSubcoreMesh(core_axis_name="c", subcore_axis_name="s")


def _span_plan(R):
    n_full = R // _SPAN
    rem = R % _SPAN
    assert n_full < _NW or (n_full == _NW and rem == 0)
    return n_full, rem


def _fill_rows(ref, nrows, ncols, value):
    vec = jnp.full((16,), value, jnp.float32)
    cols = ncols // 16

    def body(i, _):
        for j in range(cols):
            ref[i, pl.ds(j * 16, 16)] = vec
        return 0

    lax.fori_loop(0, nrows, body, 0)


def _deg_body(src2_hbm, dst2_hbm, out_hbm, dacc0, dacc1, h0, h1,
              hstage0, hstage1, sidx16, didx16, ibuf, zbuf, dsem):
    c = lax.axis_index("c")
    s = lax.axis_index("s")
    w = c * _NS + s
    N = out_hbm.shape[2]
    NP = dacc0.shape[0]
    n_full, rem = _span_plan(src2_hbm.shape[0])
    assert n_full == _NW - 1 and rem > 0 and _SPAN % _G == 0
    ngroups = jnp.where(w < n_full, _SPAN // _G, rem // _G)
    gtail = rem % _G
    rpsp = NP // _NS

    ones16 = jnp.full((16,), 1.0, jnp.float32)
    zero16 = jnp.zeros((16,), jnp.float32)

    def zh(i, carry):
        h0[pl.ds(i * 16, 16)] = zero16
        h1[pl.ds(i * 16, 16)] = zero16
        return carry

    _fill_rows(zbuf, _C, 16, 0.0)
    zds = []
    for k in range(rpsp // _C):
        base = s * rpsp + k * _C
        zds.append(pltpu.async_copy(zbuf, dacc0.at[pl.ds(base, _C)], dsem))
        zds.append(pltpu.async_copy(zbuf, dacc1.at[pl.ds(base, _C)], dsem))
    lax.fori_loop(0, NP // 16, zh, 0)
    for d in zds:
        d.wait()
    plsc.subcore_barrier()

    def group_body(rowbase, cnt):
        sd = pltpu.async_copy(src2_hbm.at[pl.ds(rowbase, cnt)],
                              sidx16.at[pl.ds(0, cnt)], dsem)
        pltpu.sync_copy(dst2_hbm.at[pl.ds(rowbase, cnt)],
                        didx16.at[pl.ds(0, cnt)])
        sd.wait()
        for u in range(cnt):
            for v in range(_C // 16):
                plsc.addupdate_scatter(h0, [sidx16[u, pl.ds(v * 16, 16)]],
                                       ones16)
                plsc.addupdate_scatter(h1, [didx16[u, pl.ds(v * 16, 16)]],
                                       ones16)

    def group(g, carry):
        group_body(w * _SPAN + g * _G, _G)
        return carry

    lax.fori_loop(0, ngroups, group, 0)
    if gtail:
        @pl.when(w == n_full)
        def _():
            group_body(n_full * _SPAN + (rem // _G) * _G, gtail)

    for m in range(NP // 16 // _C):
        cb = m * _C

        def repack(r, carry, cb=cb):
            hstage0[r, :] = h0[pl.ds((cb + r) * 16, 16)]
            hstage1[r, :] = h1[pl.ds((cb + r) * 16, 16)]
            return carry

        lax.fori_loop(0, _C, repack, 0)
        for j in range(_C // 16):
            ibuf[pl.ds(j * 16, 16)] = (lax.iota(jnp.int32, 16)
                                       + (cb + j * 16))
        pltpu.sync_copy(hstage0, dacc0.at[ibuf], add=True)
        pltpu.sync_copy(hstage1, dacc1.at[ibuf], add=True)
    plsc.subcore_barrier()
    ra = (N // _NS) & ~7
    row0 = s * ra
    pltpu.sync_copy(dacc0.at[pl.ds(row0, ra)], out_hbm.at[c, 0, pl.ds(row0, ra)])
    pltpu.sync_copy(dacc1.at[pl.ds(row0, ra)], out_hbm.at[c, 1, pl.ds(row0, ra)])
    tail = N - ra * _NS
    if tail:
        @pl.when(s == _NS - 1)
        def _():
            t0 = ra * _NS
            pltpu.sync_copy(dacc0.at[pl.ds(t0, tail)],
                            out_hbm.at[c, 0, pl.ds(t0, tail)])
            pltpu.sync_copy(dacc1.at[pl.ds(t0, tail)],
                            out_hbm.at[c, 1, pl.ds(t0, tail)])


def _mp_body(hs_hbm, src2_hbm, dst2_hbm, out_hbm, acc, r0, r1,
             sidx16, didx16, gs0, gs1):
    rows = [r0, r1]
    gsem = [gs0, gs1]
    c = lax.axis_index("c")
    s = lax.axis_index("s")
    w = c * _NS + s
    R = src2_hbm.shape[0]
    N, D = hs_hbm.shape
    n_full, rem = _span_plan(R)
    assert n_full == _NW - 1 and rem > 0
    assert _SPAN % _G == 0
    ngroups = jnp.where(w < n_full, _SPAN // _G, rem // _G)
    tail = rem % _G
    rps = N // _NS

    _fill_rows(r0, _C, D, 0.0)
    zds = [pltpu.async_copy(r0.at[pl.ds(0, _ZR)],
                            acc.at[pl.ds(s * rps + k * _ZR, _ZR)], gs0)
           for k in range(rps // _ZR)]
    for d in zds:
        d.wait()
    plsc.subcore_barrier()

    def group_body(rowbase, cnt):
        sd = pltpu.async_copy(src2_hbm.at[pl.ds(rowbase, cnt)],
                              sidx16.at[pl.ds(0, cnt)], gs0)
        pltpu.sync_copy(dst2_hbm.at[pl.ds(rowbase, cnt)],
                        didx16.at[pl.ds(0, cnt)])
        sd.wait()
        descs = {}
        for u in range(min(2, cnt)):
            descs[u] = pltpu.async_copy(
                hs_hbm.at[sidx16.at[u]], rows[u % 2], gsem[u % 2])
        for u in range(cnt):
            descs[u].wait()
            pltpu.sync_copy(rows[u % 2], acc.at[didx16.at[u]], add=True)
            if u + 2 < cnt:
                descs[u + 2] = pltpu.async_copy(
                    hs_hbm.at[sidx16.at[u + 2]], rows[u % 2], gsem[u % 2])

    def group(g, carry):
        group_body(w * _SPAN + g * _G, _G)
        return carry

    lax.fori_loop(0, ngroups, group, 0)
    if tail:
        @pl.when(w == n_full)
        def _():
            group_body(n_full * _SPAN + (rem // _G) * _G, tail)

    plsc.subcore_barrier()
    ra = (N // _NS) & ~7
    row0 = s * ra
    pltpu.sync_copy(acc.at[pl.ds(row0, ra)], out_hbm.at[c, pl.ds(row0, ra)])
    tail = N - ra * _NS
    if tail:
        @pl.when(s == _NS - 1)
        def _():
            t0 = ra * _NS
            pltpu.sync_copy(acc.at[pl.ds(t0, tail)],
                            out_hbm.at[c, pl.ds(t0, tail)])


def _sc_degrees(src2, dst2, N):
    npad = 16 * _C * ((N + 16 * _C - 1) // (16 * _C))
    k = pl.kernel(
        _deg_body,
        out_type=jax.ShapeDtypeStruct((_NC, 2, N, 16), jnp.float32),
        mesh=_sc_mesh(),
        scratch_types=[
            pltpu.VMEM_SHARED((npad, 16), jnp.float32),
            pltpu.VMEM_SHARED((npad, 16), jnp.float32),
            pltpu.VMEM((npad,), jnp.float32),
            pltpu.VMEM((npad,), jnp.float32),
            pltpu.VMEM((_C, 16), jnp.float32),
            pltpu.VMEM((_C, 16), jnp.float32),
            pltpu.VMEM((_G, _C), jnp.int32),
            pltpu.VMEM((_G, _C), jnp.int32),
            pltpu.VMEM((_C,), jnp.int32),
            pltpu.VMEM((_C, 16), jnp.float32),
            pltpu.SemaphoreType.DMA,
        ],
        compiler_params=pltpu.CompilerParams(needs_layout_passes=False),
    )
    return k(src2, dst2)


def _sc_message_pass(hs, src2, dst2):
    N, D = hs.shape
    k = pl.kernel(
        _mp_body,
        out_type=jax.ShapeDtypeStruct((_NC, N, D), jnp.float32),
        mesh=_sc_mesh(),
        scratch_types=[
            pltpu.VMEM_SHARED((N, D), jnp.float32),
            pltpu.VMEM((_C, D), jnp.float32),
            pltpu.VMEM((_C, D), jnp.float32),
            pltpu.VMEM((_G, _C), jnp.int32),
            pltpu.VMEM((_G, _C), jnp.int32),
            pltpu.SemaphoreType.DMA,
            pltpu.SemaphoreType.DMA,
        ],
    )
    return k(hs, src2, dst2)


def _dot(a, w):
    return jnp.dot(a, w, preferred_element_type=jnp.float32,
                   precision=lax.Precision.HIGHEST)


def _tc_prep_body(x_ref, mk_ref, tok_ref, do0, do1, di0, di1,
                  h1s_ref, dvo_ref, dvi_ref):
    deg_o = do0[:, :1] + do1[:, :1]
    deg_i = di0[:, :1] + di1[:, :1]
    dvo = lax.rsqrt(jnp.maximum(deg_o, 1.0))
    dvi = lax.rsqrt(jnp.maximum(deg_i, 1.0))
    m = mk_ref[...]
    ux = x_ref[...] * (1.0 - m) + m * tok_ref[...]
    shape = ux.shape
    dvo_b = jnp.broadcast_to(dvo, shape)
    h1s_ref[...] = ux * dvo_b
    dvo_ref[...] = dvo_b
    dvi_ref[...] = jnp.broadcast_to(dvi, shape)


def _tc_layer1_body(mp_ref, dvi_ref, dvo_ref, w_ref, b_ref, out_ref):
    m = (mp_ref[0] + mp_ref[1]) * dvi_ref[...]
    y = _dot(m, w_ref[...]) + b_ref[...]
    out_ref[...] = jnp.maximum(y, 0.0) * dvo_ref[...]


def _tc_layer2_body(mp_ref, dvi_ref, dvo_ref, w2_ref, b2_ref, we_ref, mk_ref,
                    out_ref):
    m = (mp_ref[0] + mp_ref[1]) * dvi_ref[...]
    h = jnp.maximum(_dot(m, w2_ref[...]) + b2_ref[...], 0.0)
    rep = _dot(h, we_ref[...])
    out_ref[...] = rep * (1.0 - mk_ref[...]) * dvo_ref[...]


def _tc_loss_body(inv_nmask, mp_ref, dvi_ref, wd_ref, bd_ref, x_ref, mk_ref,
                  out_ref):
    i = pl.program_id(0)
    m = (mp_ref[0] + mp_ref[1]) * dvi_ref[...]
    recon = _dot(m, wd_ref[...]) + bd_ref[...]
    u = x_ref[...]
    duv = jnp.sum(u * recon, axis=1, keepdims=True)
    nu = jnp.sqrt(jnp.sum(u * u, axis=1, keepdims=True))
    nv = jnp.sqrt(jnp.sum(recon * recon, axis=1, keepdims=True))
    cos = duv / ((nu + 1e-8) * (nv + 1e-8))
    t = 1.0 - cos
    part = jnp.sum(t * t * mk_ref[:, :1]) * inv_nmask

    @pl.when(i == 0)
    def _():
        out_ref[0, 0] = 0.0

    out_ref[0, 0] += part


def _row_spec(B, w):
    return pl.BlockSpec((B, w), lambda i: (i, 0))


def _const_spec(shape):
    return pl.BlockSpec(shape, lambda i: tuple(0 for _ in shape))


def kernel(x, edge_index, W_enc1, b_enc1, W_enc2, b_enc2, W_e2d, W_dec, b_dec,
           mask_token):
    N, D = x.shape
    E = edge_index.shape[1]
    src1 = edge_index[0]
    dst1 = edge_index[1]
    src2 = src1.reshape(E // _C, _C)
    dst2 = dst1.reshape(E // _C, _C)

    num_mask = int(_MASK_RATE * N)
    mask_nodes = jax.random.permutation(jax.random.key(42), N)[:num_mask]
    maskf = jnp.zeros((N, 1), jnp.float32).at[mask_nodes].set(1.0)
    maskf = jnp.asarray(jnp.broadcast_to(maskf, (N, D)))

    b1 = b_enc1.reshape(1, D)
    b2 = b_enc2.reshape(1, D)
    bd = b_dec.reshape(1, D)

    B = 2000
    grid = (N // B,)
    mp_spec = pl.BlockSpec((_NC, B, D), lambda i: (0, i, 0))

    degp = _sc_degrees(src2, dst2, N)

    h1s, dvo, dvi = pl.pallas_call(
        _tc_prep_body,
        grid=grid,
        in_specs=[
            _row_spec(B, D), _row_spec(B, D), _const_spec((1, D)),
            _row_spec(B, 16), _row_spec(B, 16),
            _row_spec(B, 16), _row_spec(B, 16),
        ],
        out_specs=[_row_spec(B, D)] * 3,
        out_shape=[jax.ShapeDtypeStruct((N, D), jnp.float32)] * 3,
    )(x, maskf, mask_token, degp[0, 0], degp[1, 0], degp[0, 1], degp[1, 1])

    mp1 = _sc_message_pass(h1s, src2, dst2)

    h2s = pl.pallas_call(
        _tc_layer1_body,
        grid=grid,
        in_specs=[mp_spec, _row_spec(B, D), _row_spec(B, D),
                  _const_spec((D, D)), _const_spec((1, D))],
        out_specs=_row_spec(B, D),
        out_shape=jax.ShapeDtypeStruct((N, D), jnp.float32),
    )(mp1, dvi, dvo, W_enc1, b1)

    mp2 = _sc_message_pass(h2s, src2, dst2)

    h3s = pl.pallas_call(
        _tc_layer2_body,
        grid=grid,
        in_specs=[mp_spec, _row_spec(B, D), _row_spec(B, D),
                  _const_spec((D, D)), _const_spec((1, D)),
                  _const_spec((D, D)), _row_spec(B, D)],
        out_specs=_row_spec(B, D),
        out_shape=jax.ShapeDtypeStruct((N, D), jnp.float32),
    )(mp2, dvi, dvo, W_enc2, b2, W_e2d, maskf)

    mp3 = _sc_message_pass(h3s, src2, dst2)

    out = pl.pallas_call(
        functools.partial(_tc_loss_body, 1.0 / num_mask),
        grid=grid,
        in_specs=[mp_spec, _row_spec(B, D), _const_spec((D, D)),
                  _const_spec((1, D)), _row_spec(B, D), _row_spec(B, D)],
        out_specs=pl.BlockSpec(memory_space=pltpu.SMEM),
        out_shape=jax.ShapeDtypeStruct((1, 1), jnp.float32),
    )(mp3, dvi, W_dec, bd, x, maskf)

    return out[0, 0]


# speedup vs baseline: 1.0449x; 1.0008x over previous
import functools

import jax
import jax.numpy as jnp
from jax import lax
from jax.experimental import pallas as pl
from jax.experimental.pallas import tpu as pltpu
from jax.experimental.pallas import tpu_sc as plsc

_MASK_RATE = 0.3
_NC = 2
_NS = 16
_NW = _NC * _NS
_C = 128
_SPAN = 80
_ZR = 125
_G = 16


def _sc_mesh():
    return plsc.VectorSubcoreMesh(core_axis_name="c", subcore_axis_name="s")


def _span_plan(R):
    n_full = R // _SPAN
    rem = R % _SPAN
    assert n_full < _NW or (n_full == _NW and rem == 0)
    return n_full, rem


def _fill_rows(ref, nrows, ncols, value):
    vec = jnp.full((16,), value, jnp.float32)
    cols = ncols // 16

    def body(i, _):
        for j in range(cols):
            ref[i, pl.ds(j * 16, 16)] = vec
        return 0

    lax.fori_loop(0, nrows, body, 0)


def _deg_body(src2_hbm, dst2_hbm, out_hbm, dacc0, dacc1, h0, h1,
              hstage0, hstage1, sidx16, didx16, ibuf, zbuf, dsem):
    c = lax.axis_index("c")
    s = lax.axis_index("s")
    w = c * _NS + s
    N = out_hbm.shape[2]
    NP = dacc0.shape[0]
    n_full, rem = _span_plan(src2_hbm.shape[0])
    assert n_full == _NW - 1 and rem > 0 and _SPAN % _G == 0
    ngroups = jnp.where(w < n_full, _SPAN // _G, rem // _G)
    gtail = rem % _G
    rpsp = NP // _NS

    ones16 = jnp.full((16,), 1.0, jnp.float32)
    zero16 = jnp.zeros((16,), jnp.float32)

    def zh(i, carry):
        h0[pl.ds(i * 16, 16)] = zero16
        h1[pl.ds(i * 16, 16)] = zero16
        return carry

    _fill_rows(zbuf, _C, 16, 0.0)
    zds = []
    for k in range(rpsp // _C):
        base = s * rpsp + k * _C
        zds.append(pltpu.async_copy(zbuf, dacc0.at[pl.ds(base, _C)], dsem))
        zds.append(pltpu.async_copy(zbuf, dacc1.at[pl.ds(base, _C)], dsem))
    lax.fori_loop(0, NP // 16, zh, 0)
    for d in zds:
        d.wait()
    plsc.subcore_barrier()

    def group_body(rowbase, cnt):
        sd = pltpu.async_copy(src2_hbm.at[pl.ds(rowbase, cnt)],
                              sidx16.at[pl.ds(0, cnt)], dsem)
        pltpu.sync_copy(dst2_hbm.at[pl.ds(rowbase, cnt)],
                        didx16.at[pl.ds(0, cnt)])
        sd.wait()
        for u in range(cnt):
            for v in range(_C // 16):
                plsc.addupdate_scatter(h0, [sidx16[u, pl.ds(v * 16, 16)]],
                                       ones16)
                plsc.addupdate_scatter(h1, [didx16[u, pl.ds(v * 16, 16)]],
                                       ones16)

    def group(g, carry):
        group_body(w * _SPAN + g * _G, _G)
        return carry

    lax.fori_loop(0, ngroups, group, 0)
    if gtail:
        @pl.when(w == n_full)
        def _():
            group_body(n_full * _SPAN + (rem // _G) * _G, gtail)

    for m in range(NP // 16 // _C):
        cb = m * _C

        def repack(r, carry, cb=cb):
            hstage0[r, :] = h0[pl.ds((cb + r) * 16, 16)]
            hstage1[r, :] = h1[pl.ds((cb + r) * 16, 16)]
            return carry

        lax.fori_loop(0, _C, repack, 0)
        for j in range(_C // 16):
            ibuf[pl.ds(j * 16, 16)] = (lax.iota(jnp.int32, 16)
                                       + (cb + j * 16))
        pltpu.sync_copy(hstage0, dacc0.at[ibuf], add=True)
        pltpu.sync_copy(hstage1, dacc1.at[ibuf], add=True)
    plsc.subcore_barrier()
    ra = (N // _NS) & ~7
    row0 = s * ra
    pltpu.sync_copy(dacc0.at[pl.ds(row0, ra)], out_hbm.at[c, 0, pl.ds(row0, ra)])
    pltpu.sync_copy(dacc1.at[pl.ds(row0, ra)], out_hbm.at[c, 1, pl.ds(row0, ra)])
    tail = N - ra * _NS
    if tail:
        @pl.when(s == _NS - 1)
        def _():
            t0 = ra * _NS
            pltpu.sync_copy(dacc0.at[pl.ds(t0, tail)],
                            out_hbm.at[c, 0, pl.ds(t0, tail)])
            pltpu.sync_copy(dacc1.at[pl.ds(t0, tail)],
                            out_hbm.at[c, 1, pl.ds(t0, tail)])


def _mp_body(hs_hbm, src2_hbm, dst2_hbm, out_hbm, acc, r0, r1,
             sidx16, didx16, gs0, gs1, isem):
    rows = [r0, r1]
    gsem = [gs0, gs1]
    c = lax.axis_index("c")
    s = lax.axis_index("s")
    w = c * _NS + s
    R = src2_hbm.shape[0]
    N, D = hs_hbm.shape
    n_full, rem = _span_plan(R)
    assert n_full == _NW - 1 and rem > 0
    assert _SPAN % _G == 0
    ngroups = jnp.where(w < n_full, _SPAN // _G, rem // _G)
    tail = rem % _G
    rps = N // _NS

    _fill_rows(r0, _C, D, 0.0)
    zds = [pltpu.async_copy(r0.at[pl.ds(0, _ZR)],
                            acc.at[pl.ds(s * rps + k * _ZR, _ZR)], gs0)
           for k in range(rps // _ZR)]
    for d in zds:
        d.wait()
    plsc.subcore_barrier()

    def group_body(rowbase, cnt):
        sd = pltpu.async_copy(src2_hbm.at[pl.ds(rowbase, cnt)],
                              sidx16.at[pl.ds(0, cnt)], gs0)
        dd = pltpu.async_copy(dst2_hbm.at[pl.ds(rowbase, cnt)],
                              didx16.at[pl.ds(0, cnt)], isem)
        sd.wait()
        descs = {}
        for u in range(min(2, cnt)):
            descs[u] = pltpu.async_copy(
                hs_hbm.at[sidx16.at[u]], rows[u % 2], gsem[u % 2])
        dd.wait()
        for u in range(cnt):
            descs[u].wait()
            pltpu.sync_copy(rows[u % 2], acc.at[didx16.at[u]], add=True)
            if u + 2 < cnt:
                descs[u + 2] = pltpu.async_copy(
                    hs_hbm.at[sidx16.at[u + 2]], rows[u % 2], gsem[u % 2])

    def group(g, carry):
        group_body(w * _SPAN + g * _G, _G)
        return carry

    lax.fori_loop(0, ngroups, group, 0)
    if tail:
        @pl.when(w == n_full)
        def _():
            group_body(n_full * _SPAN + (rem // _G) * _G, tail)

    plsc.subcore_barrier()
    ra = (N // _NS) & ~7
    row0 = s * ra
    pltpu.sync_copy(acc.at[pl.ds(row0, ra)], out_hbm.at[c, pl.ds(row0, ra)])
    tail = N - ra * _NS
    if tail:
        @pl.when(s == _NS - 1)
        def _():
            t0 = ra * _NS
            pltpu.sync_copy(acc.at[pl.ds(t0, tail)],
                            out_hbm.at[c, pl.ds(t0, tail)])


def _sc_degrees(src2, dst2, N):
    npad = 16 * _C * ((N + 16 * _C - 1) // (16 * _C))
    k = pl.kernel(
        _deg_body,
        out_type=jax.ShapeDtypeStruct((_NC, 2, N, 16), jnp.float32),
        mesh=_sc_mesh(),
        scratch_types=[
            pltpu.VMEM_SHARED((npad, 16), jnp.float32),
            pltpu.VMEM_SHARED((npad, 16), jnp.float32),
            pltpu.VMEM((npad,), jnp.float32),
            pltpu.VMEM((npad,), jnp.float32),
            pltpu.VMEM((_C, 16), jnp.float32),
            pltpu.VMEM((_C, 16), jnp.float32),
            pltpu.VMEM((_G, _C), jnp.int32),
            pltpu.VMEM((_G, _C), jnp.int32),
            pltpu.VMEM((_C,), jnp.int32),
            pltpu.VMEM((_C, 16), jnp.float32),
            pltpu.SemaphoreType.DMA,
        ],
        compiler_params=pltpu.CompilerParams(needs_layout_passes=False),
    )
    return k(src2, dst2)


def _sc_message_pass(hs, src2, dst2):
    N, D = hs.shape
    k = pl.kernel(
        _mp_body,
        out_type=jax.ShapeDtypeStruct((_NC, N, D), jnp.float32),
        mesh=_sc_mesh(),
        scratch_types=[
            pltpu.VMEM_SHARED((N, D), jnp.float32),
            pltpu.VMEM((_C, D), jnp.float32),
            pltpu.VMEM((_C, D), jnp.float32),
            pltpu.VMEM((_G, _C), jnp.int32),
            pltpu.VMEM((_G, _C), jnp.int32),
            pltpu.SemaphoreType.DMA,
            pltpu.SemaphoreType.DMA,
            pltpu.SemaphoreType.DMA,
        ],
    )
    return k(hs, src2, dst2)


def _dot(a, w):
    return jnp.dot(a, w, preferred_element_type=jnp.float32,
                   precision=lax.Precision.HIGHEST)


def _tc_prep_body(x_ref, mk_ref, tok_ref, do0, do1, di0, di1,
                  h1s_ref, dvo_ref, dvi_ref):
    deg_o = do0[:, :1] + do1[:, :1]
    deg_i = di0[:, :1] + di1[:, :1]
    dvo = lax.rsqrt(jnp.maximum(deg_o, 1.0))
    dvi = lax.rsqrt(jnp.maximum(deg_i, 1.0))
    m = mk_ref[...]
    ux = x_ref[...] * (1.0 - m) + m * tok_ref[...]
    shape = ux.shape
    dvo_b = jnp.broadcast_to(dvo, shape)
    h1s_ref[...] = ux * dvo_b
    dvo_ref[...] = dvo_b
    dvi_ref[...] = jnp.broadcast_to(dvi, shape)


def _tc_layer1_body(mp_ref, dvi_ref, dvo_ref, w_ref, b_ref, out_ref):
    m = (mp_ref[0] + mp_ref[1]) * dvi_ref[...]
    y = _dot(m, w_ref[...]) + b_ref[...]
    out_ref[...] = jnp.maximum(y, 0.0) * dvo_ref[...]


def _tc_layer2_body(mp_ref, dvi_ref, dvo_ref, w2_ref, b2_ref, we_ref, mk_ref,
                    out_ref):
    m = (mp_ref[0] + mp_ref[1]) * dvi_ref[...]
    h = jnp.maximum(_dot(m, w2_ref[...]) + b2_ref[...], 0.0)
    rep = _dot(h, we_ref[...])
    out_ref[...] = rep * (1.0 - mk_ref[...]) * dvo_ref[...]


def _tc_loss_body(inv_nmask, mp_ref, dvi_ref, wd_ref, bd_ref, x_ref, mk_ref,
                  out_ref):
    i = pl.program_id(0)
    m = (mp_ref[0] + mp_ref[1]) * dvi_ref[...]
    recon = _dot(m, wd_ref[...]) + bd_ref[...]
    u = x_ref[...]
    duv = jnp.sum(u * recon, axis=1, keepdims=True)
    nu = jnp.sqrt(jnp.sum(u * u, axis=1, keepdims=True))
    nv = jnp.sqrt(jnp.sum(recon * recon, axis=1, keepdims=True))
    cos = duv / ((nu + 1e-8) * (nv + 1e-8))
    t = 1.0 - cos
    part = jnp.sum(t * t * mk_ref[:, :1]) * inv_nmask

    @pl.when(i == 0)
    def _():
        out_ref[0, 0] = 0.0

    out_ref[0, 0] += part


def _row_spec(B, w):
    return pl.BlockSpec((B, w), lambda i: (i, 0))


def _const_spec(shape):
    return pl.BlockSpec(shape, lambda i: tuple(0 for _ in shape))


def kernel(x, edge_index, W_enc1, b_enc1, W_enc2, b_enc2, W_e2d, W_dec, b_dec,
           mask_token):
    N, D = x.shape
    E = edge_index.shape[1]
    src1 = edge_index[0]
    dst1 = edge_index[1]
    src2 = src1.reshape(E // _C, _C)
    dst2 = dst1.reshape(E // _C, _C)

    num_mask = int(_MASK_RATE * N)
    mask_nodes = jax.random.permutation(jax.random.key(42), N)[:num_mask]
    maskf = jnp.zeros((N, 1), jnp.float32).at[mask_nodes].set(1.0)
    maskf = jnp.asarray(jnp.broadcast_to(maskf, (N, D)))

    b1 = b_enc1.reshape(1, D)
    b2 = b_enc2.reshape(1, D)
    bd = b_dec.reshape(1, D)

    B = 2000
    grid = (N // B,)
    mp_spec = pl.BlockSpec((_NC, B, D), lambda i: (0, i, 0))

    degp = _sc_degrees(src2, dst2, N)

    h1s, dvo, dvi = pl.pallas_call(
        _tc_prep_body,
        grid=grid,
        in_specs=[
            _row_spec(B, D), _row_spec(B, D), _const_spec((1, D)),
            _row_spec(B, 16), _row_spec(B, 16),
            _row_spec(B, 16), _row_spec(B, 16),
        ],
        out_specs=[_row_spec(B, D)] * 3,
        out_shape=[jax.ShapeDtypeStruct((N, D), jnp.float32)] * 3,
    )(x, maskf, mask_token, degp[0, 0], degp[1, 0], degp[0, 1], degp[1, 1])

    mp1 = _sc_message_pass(h1s, src2, dst2)

    h2s = pl.pallas_call(
        _tc_layer1_body,
        grid=grid,
        in_specs=[mp_spec, _row_spec(B, D), _row_spec(B, D),
                  _const_spec((D, D)), _const_spec((1, D))],
        out_specs=_row_spec(B, D),
        out_shape=jax.ShapeDtypeStruct((N, D), jnp.float32),
    )(mp1, dvi, dvo, W_enc1, b1)

    mp2 = _sc_message_pass(h2s, src2, dst2)

    h3s = pl.pallas_call(
        _tc_layer2_body,
        grid=grid,
        in_specs=[mp_spec, _row_spec(B, D), _row_spec(B, D),
                  _const_spec((D, D)), _const_spec((1, D)),
                  _const_spec((D, D)), _row_spec(B, D)],
        out_specs=_row_spec(B, D),
        out_shape=jax.ShapeDtypeStruct((N, D), jnp.float32),
    )(mp2, dvi, dvo, W_enc2, b2, W_e2d, maskf)

    mp3 = _sc_message_pass(h3s, src2, dst2)

    out = pl.pallas_call(
        functools.partial(_tc_loss_body, 1.0 / num_mask),
        grid=grid,
        in_specs=[mp_spec, _row_spec(B, D), _const_spec((D, D)),
                  _const_spec((1, D)), _row_spec(B, D), _row_spec(B, D)],
        out_specs=pl.BlockSpec(memory_space=pltpu.SMEM),
        out_shape=jax.ShapeDtypeStruct((1, 1), jnp.float32),
    )(mp3, dvi, W_dec, bd, x, maskf)

    return out[0, 0]
